# baseline (speedup 1.0000x reference)
import functools

import jax
import jax.numpy as jnp
from jax import lax
from jax.experimental import pallas as pl
from jax.experimental.pallas import tpu as pltpu
from jax.experimental.pallas import tpu_sc as plsc

_BATCH = 16384
_TILES = 16
_IDS_PER_TILE = _BATCH // _TILES


def _gather_body(rgb_hbm, ir_hbm, table_v_hbm, table_r_hbm,
                 out_v_hbm, out_r_hbm,
                 idx_v, idx_r, rows_v, rows_r,
                 sem_iv, sem_gv, sem_ov, sem_ir, sem_gr, sem_or):
    cid = lax.axis_index("c")
    sl = pl.ds(lax.axis_index("s") * _IDS_PER_TILE, _IDS_PER_TILE)

    @pl.when(cid == 0)
    def _():
        pltpu.async_copy(rgb_hbm.at[sl], idx_v, sem_iv).wait()
        pltpu.async_copy(table_v_hbm.at[idx_v], rows_v, sem_gv).wait()
        pltpu.async_copy(rows_v, out_v_hbm.at[sl], sem_ov).wait()

    @pl.when(cid == 1)
    def _():
        pltpu.async_copy(ir_hbm.at[sl], idx_r, sem_ir).wait()
        pltpu.async_copy(table_r_hbm.at[idx_r], rows_r, sem_gr).wait()
        pltpu.async_copy(rows_r, out_r_hbm.at[sl], sem_or).wait()


@jax.jit
def kernel(rgb_ids, ir_ids, class_confidence_v, class_confidence_r):
    mesh = plsc.VectorSubcoreMesh(core_axis_name="c", subcore_axis_name="s")
    f = functools.partial(
        pl.kernel,
        mesh=mesh,
        out_type=(
            jax.ShapeDtypeStruct((_BATCH,), jnp.float32),
            jax.ShapeDtypeStruct((_BATCH,), jnp.float32),
        ),
        scratch_types=[
            pltpu.VMEM((_IDS_PER_TILE,), jnp.int32),
            pltpu.VMEM((_IDS_PER_TILE,), jnp.int32),
            pltpu.VMEM((_IDS_PER_TILE,), jnp.float32),
            pltpu.VMEM((_IDS_PER_TILE,), jnp.float32),
            pltpu.SemaphoreType.DMA,
            pltpu.SemaphoreType.DMA,
            pltpu.SemaphoreType.DMA,
            pltpu.SemaphoreType.DMA,
            pltpu.SemaphoreType.DMA,
            pltpu.SemaphoreType.DMA,
        ],
    )(_gather_body)
    return f(rgb_ids.astype(jnp.int32), ir_ids.astype(jnp.int32),
             class_confidence_v, class_confidence_r)


# speedup vs baseline: 1.0254x; 1.0254x over previous
import functools

import jax
import jax.numpy as jnp
from jax import lax
from jax.experimental import pallas as pl
from jax.experimental.pallas import tpu as pltpu
from jax.experimental.pallas import tpu_sc as plsc

_BATCH = 16384
_NUM_WORKERS = 32
_IDS_PER_WORKER = _BATCH // _NUM_WORKERS
_CHUNKS = 2
_CHUNK = _IDS_PER_WORKER // _CHUNKS


def _gather_body(rgb_hbm, ir_hbm, table_v_hbm, table_r_hbm,
                 out_v_hbm, out_r_hbm,
                 idx_v, idx_r, rows_v, rows_r, sem_i, sem_gv, sem_gr, sem_o):
    wid = lax.axis_index("s") * 2 + lax.axis_index("c")
    sl = pl.ds(wid * _IDS_PER_WORKER, _IDS_PER_WORKER)
    cp_iv = pltpu.async_copy(rgb_hbm.at[sl], idx_v, sem_i)
    cp_ir = pltpu.async_copy(ir_hbm.at[sl], idx_r, sem_i)
    cp_iv.wait()
    gv = []
    for c in range(_CHUNKS):
        csl = pl.ds(c * _CHUNK, _CHUNK)
        gv.append(pltpu.async_copy(
            table_v_hbm.at[idx_v.at[csl]], rows_v.at[csl], sem_gv))
    cp_ir.wait()
    gr = []
    for c in range(_CHUNKS):
        csl = pl.ds(c * _CHUNK, _CHUNK)
        gr.append(pltpu.async_copy(
            table_r_hbm.at[idx_r.at[csl]], rows_r.at[csl], sem_gr))
    for cp in gv:
        cp.wait()
    cp_ov = pltpu.async_copy(rows_v, out_v_hbm.at[sl], sem_o)
    for cp in gr:
        cp.wait()
    cp_or = pltpu.async_copy(rows_r, out_r_hbm.at[sl], sem_o)
    cp_ov.wait()
    cp_or.wait()


@jax.jit
def kernel(rgb_ids, ir_ids, class_confidence_v, class_confidence_r):
    mesh = plsc.VectorSubcoreMesh(core_axis_name="c", subcore_axis_name="s")
    f = functools.partial(
        pl.kernel,
        mesh=mesh,
        out_type=(
            jax.ShapeDtypeStruct((_BATCH,), jnp.float32),
            jax.ShapeDtypeStruct((_BATCH,), jnp.float32),
        ),
        scratch_types=[
            pltpu.VMEM((_IDS_PER_WORKER,), jnp.int32),
            pltpu.VMEM((_IDS_PER_WORKER,), jnp.int32),
            pltpu.VMEM((_IDS_PER_WORKER,), jnp.float32),
            pltpu.VMEM((_IDS_PER_WORKER,), jnp.float32),
            pltpu.SemaphoreType.DMA,
            pltpu.SemaphoreType.DMA,
            pltpu.SemaphoreType.DMA,
            pltpu.SemaphoreType.DMA,
        ],
    )(_gather_body)
    return f(rgb_ids.astype(jnp.int32), ir_ids.astype(jnp.int32),
             class_confidence_v, class_confidence_r)
